# baseline (speedup 1.0000x reference)
import functools

import jax
import jax.numpy as jnp
from jax import lax
from jax.experimental import pallas as pl
from jax.experimental.pallas import tpu as pltpu
from jax.experimental.pallas import tpu_sc as plsc

F32 = jnp.float32
PW = 128


def _softplus(x):
    return jnp.maximum(x, 0.0) + jnp.log(1.0 + jnp.exp(-jnp.abs(x)))


def _sigmoid(x):
    return 1.0 / (1.0 + jnp.exp(-x))


def _dot(x, w):
    return lax.dot_general(x, w, (((1,), (0,)), ((), ())),
                           preferred_element_type=F32)


def _padw(x, T):
    K = x.shape[1]
    return jnp.concatenate([x, jnp.zeros((T, PW - K), F32)], axis=1)


def _sc_gather(table, idx):
    V, D = table.shape
    B = idx.shape[0]
    info = plsc.get_sparse_core_info()
    ncores = info.num_cores
    nw = ncores * info.num_subcores
    bw = B // nw
    ch = 200 if bw % 200 == 0 else bw
    nchunk = bw // ch
    npair = (nchunk - 1) // 2
    tail = nchunk - 1 - 2 * npair
    mesh = plsc.VectorSubcoreMesh(core_axis_name="c", subcore_axis_name="s")

    @functools.partial(
        pl.kernel, mesh=mesh,
        out_type=jax.ShapeDtypeStruct((B, D), F32),
        scratch_types=[pltpu.VMEM((ch,), jnp.int32),
                       pltpu.VMEM((ch,), jnp.int32),
                       pltpu.VMEM((ch, D), F32),
                       pltpu.VMEM((ch, D), F32),
                       pltpu.SemaphoreType.DMA,
                       pltpu.SemaphoreType.DMA],
    )
    def k(table_hbm, idx_hbm, out_hbm, idx0, idx1, rows0, rows1, sem0, sem1):
        wid = lax.axis_index("s") * ncores + lax.axis_index("c")
        base = wid * bw

        def start(c, idx_v, rows_v, sem):
            pltpu.sync_copy(idx_hbm.at[pl.ds(base + c * ch, ch)], idx_v)
            pltpu.async_copy(table_hbm.at[idx_v], rows_v, sem)

        def drain(c, rows_v, sem):
            pltpu.make_async_copy(table_hbm.at[pl.ds(0, ch)], rows_v,
                                  sem).wait()
            pltpu.sync_copy(rows_v, out_hbm.at[pl.ds(base + c * ch, ch)])

        start(0, idx0, rows0, sem0)

        def body(p, carry):
            c1 = 2 * p + 1
            start(c1, idx1, rows1, sem1)
            drain(c1 - 1, rows0, sem0)
            start(c1 + 1, idx0, rows0, sem0)
            drain(c1, rows1, sem1)
            return carry

        lax.fori_loop(0, npair, body, 0)
        if tail:
            start(nchunk - 1, idx1, rows1, sem1)
            drain(nchunk - 2, rows0, sem0)
            drain(nchunk - 1, rows1, sem1)
        else:
            drain(nchunk - 1, rows0, sem0)

    return k(table, idx)


_ARB = pltpu.CompilerParams(dimension_semantics=("arbitrary",))


def _full_spec(shape):
    rank = len(shape)
    return pl.BlockSpec(shape, lambda i, _r=rank: (0,) * _r)


def _embed(atom_fea, nbr_fea, WeaT, bea, WebT, beb, T):
    N, DA = atom_fea.shape
    _, M, DB = nbr_fea.shape
    A = WeaT.shape[1]
    Bb = WebT.shape[1]
    G = N // T

    def body(af_ref, nbr_ref, wa_ref, ba_ref, wb_ref, bb_ref, a_out, nb_out):
        a_out[...] = _padw(_dot(af_ref[...], wa_ref[...]) + ba_ref[...], T)
        nbf = nbr_ref[...].reshape(T * M, DB)
        nb = _dot(nbf, wb_ref[...]) + bb_ref[...]
        nb_out[...] = nb.reshape(T, M, Bb)

    return pl.pallas_call(
        body,
        grid=(G,),
        in_specs=[
            pl.BlockSpec((T, DA), lambda i: (i, 0)),
            pl.BlockSpec((T, M, DB), lambda i: (i, 0, 0)),
            _full_spec(WeaT.shape),
            _full_spec(bea.shape),
            _full_spec(WebT.shape),
            _full_spec(beb.shape),
        ],
        out_specs=[
            pl.BlockSpec((T, PW), lambda i: (i, 0)),
            pl.BlockSpec((T, M, Bb), lambda i: (i, 0, 0)),
        ],
        out_shape=[
            jax.ShapeDtypeStruct((N, PW), F32),
            jax.ShapeDtypeStruct((N, M, Bb), F32),
        ],
        compiler_params=_ARB,
    )(atom_fea, nbr_fea, WeaT, bea, WebT, beb)


def _assemble(base, gath, extra, T):
    N, Dbase = base.shape
    _, M, Dx = extra.shape
    Dout = 2 * Dbase + Dx
    G = N // T

    def body(b_ref, g_ref, x_ref, out_ref):
        b = b_ref[...]
        p0 = jnp.broadcast_to(b[:, None, :], (T, M, Dbase))
        p1 = g_ref[...].reshape(T, M, PW)[:, :, :Dbase]
        out_ref[...] = jnp.concatenate([p0, p1, x_ref[...]], axis=2)

    return pl.pallas_call(
        body,
        grid=(G,),
        in_specs=[
            pl.BlockSpec((T, Dbase), lambda i: (i, 0)),
            pl.BlockSpec((T * M, PW), lambda i: (i, 0)),
            pl.BlockSpec((T, M, Dx), lambda i: (i, 0, 0)),
        ],
        out_specs=pl.BlockSpec((T, M, Dout), lambda i: (i, 0, 0)),
        out_shape=jax.ShapeDtypeStruct((N, M, Dout), F32),
        compiler_params=_ARB,
    )(base, gath, extra)


def _conv_passA(a, an, nb, W, T):
    N = a.shape[0]
    _, M, Bb = nb.shape
    A = W[0].shape[0]
    P = W[0].shape[1]
    G = N // T

    def body(a_ref, an_ref, nb_ref, wa, wn, wb, s1, s2):
        i = pl.program_id(0)
        base = _dot(a_ref[...][:, :A], wa[...])
        baseb = jnp.broadcast_to(base[:, None, :], (T, M, P)).reshape(T * M, P)
        an_t = an_ref[...][:, :A]
        nbf = nb_ref[...].reshape(T * M, Bb)
        raw = baseb + _dot(an_t, wn[...]) + _dot(nbf, wb[...])

        @pl.when(i == 0)
        def _():
            s1[...] = jnp.zeros_like(s1)
            s2[...] = jnp.zeros_like(s2)

        s1[...] += jnp.sum(raw, axis=0, keepdims=True)
        s2[...] += jnp.sum(raw * raw, axis=0, keepdims=True)

    return pl.pallas_call(
        body,
        grid=(G,),
        in_specs=[
            pl.BlockSpec((T, PW), lambda i: (i, 0)),
            pl.BlockSpec((T * M, PW), lambda i: (i, 0)),
            pl.BlockSpec((T, M, Bb), lambda i: (i, 0, 0)),
        ] + [_full_spec(w.shape) for w in W],
        out_specs=[_full_spec((1, P)), _full_spec((1, P))],
        out_shape=[jax.ShapeDtypeStruct((1, P), F32),
                   jax.ShapeDtypeStruct((1, P), F32)],
        compiler_params=_ARB,
    )(a, an, nb, *W)


def _conv_passB(a, an, nb, W, stats, T):
    N = a.shape[0]
    _, M, Bb = nb.shape
    A = W[0].shape[0]
    P = W[0].shape[1]
    G = N // T
    R = float(N * M)

    def body(a_ref, an_ref, nb_ref, wa, wn, wb, s1, s2,
             nb_out, asum_out, s1a, s2a):
        i = pl.program_id(0)
        base = _dot(a_ref[...][:, :A], wa[...])
        baseb = jnp.broadcast_to(base[:, None, :], (T, M, P)).reshape(T * M, P)
        an_t = an_ref[...][:, :A]
        nb_t = nb_ref[...].reshape(T * M, Bb)
        raw = baseb + _dot(an_t, wn[...]) + _dot(nb_t, wb[...])

        m = s1[...] / R
        v = s2[...] / R - m * m
        rawn = (raw - m) * lax.rsqrt(v + 1e-5)

        g = _sigmoid(rawn[:, :A]) * _softplus(rawn[:, A:2 * A])
        asum_t = jnp.sum(g.reshape(T, M, A), axis=1)
        asum_out[...] = asum_t
        h = _sigmoid(rawn[:, 2 * A:2 * A + Bb]) * _softplus(rawn[:, 2 * A + Bb:])
        nb_out[...] = _softplus(nb_t + h).reshape(T, M, Bb)

        @pl.when(i == 0)
        def _():
            s1a[...] = jnp.zeros_like(s1a)
            s2a[...] = jnp.zeros_like(s2a)

        s1a[...] += jnp.sum(asum_t, axis=0, keepdims=True)
        s2a[...] += jnp.sum(asum_t * asum_t, axis=0, keepdims=True)

    return pl.pallas_call(
        body,
        grid=(G,),
        in_specs=[
            pl.BlockSpec((T, PW), lambda i: (i, 0)),
            pl.BlockSpec((T * M, PW), lambda i: (i, 0)),
            pl.BlockSpec((T, M, Bb), lambda i: (i, 0, 0)),
        ] + [_full_spec(w.shape) for w in W]
          + [_full_spec(s.shape) for s in stats],
        out_specs=[
            pl.BlockSpec((T, M, Bb), lambda i: (i, 0, 0)),
            pl.BlockSpec((T, A), lambda i: (i, 0)),
            _full_spec((1, A)),
            _full_spec((1, A)),
        ],
        out_shape=[
            jax.ShapeDtypeStruct((N, M, Bb), F32),
            jax.ShapeDtypeStruct((N, A), F32),
            jax.ShapeDtypeStruct((1, A), F32),
            jax.ShapeDtypeStruct((1, A), F32),
        ],
        compiler_params=_ARB,
    )(a, an, nb, *W, *stats)


def _conv_passC(a, asum, s1a, s2a, T):
    N = a.shape[0]
    A = asum.shape[1]
    G = N // T
    R = float(N)

    def body(a_ref, as_ref, s1_ref, s2_ref, out_ref):
        m = s1_ref[...] / R
        v = s2_ref[...] / R - m * m
        out = _softplus(a_ref[...][:, :A]
                        + (as_ref[...] - m) * lax.rsqrt(v + 1e-5))
        out_ref[...] = _padw(out, T)

    return pl.pallas_call(
        body,
        grid=(G,),
        in_specs=[
            pl.BlockSpec((T, PW), lambda i: (i, 0)),
            pl.BlockSpec((T, A), lambda i: (i, 0)),
            _full_spec((1, A)),
            _full_spec((1, A)),
        ],
        out_specs=pl.BlockSpec((T, PW), lambda i: (i, 0)),
        out_shape=jax.ShapeDtypeStruct((N, PW), F32),
        compiler_params=_ARB,
    )(a, asum, s1a, s2a)


def _pool_a(a, APC, A):
    N = a.shape[0]
    C = N // APC

    def body(a_ref, ap_out):
        at = a_ref[...][:, :A].reshape(C, APC, A)
        ap_out[...] = _softplus(jnp.mean(at, axis=1))

    return pl.pallas_call(
        body,
        grid=(1,),
        in_specs=[_full_spec((N, PW))],
        out_specs=_full_spec((C, A)),
        out_shape=jax.ShapeDtypeStruct((C, A), F32),
        compiler_params=_ARB,
    )(a)


def _pool_nb(nb, APC, TCc):
    N, M, Bb = nb.shape
    C = N // APC
    nb4 = nb.reshape(C, APC, M, Bb)
    G = C // TCc

    def body(nb_ref, nbp_out):
        nbp_out[...] = _softplus(jnp.mean(nb_ref[...], axis=1))

    return pl.pallas_call(
        body,
        grid=(G,),
        in_specs=[pl.BlockSpec((TCc, APC, M, Bb), lambda i: (i, 0, 0, 0))],
        out_specs=pl.BlockSpec((TCc, M, Bb), lambda i: (i, 0, 0)),
        out_shape=jax.ShapeDtypeStruct((C, M, Bb), F32),
        compiler_params=_ARB,
    )(nb4)


def _vae(ap, nbp, WmuaT, Wmun, bmu, WlvaT, Wlvn, blv,
         WdecaT, Wdecn, bdeca, bdecn, eps):
    C, A = ap.shape
    _, M, Bb = nbp.shape
    L = WmuaT.shape[1]

    def body(ap_ref, nbp_ref, wmua, wmun, bmu_ref, wlva, wlvn, blv_ref,
             wdeca, wdecn, bdeca_ref, bdecn_ref, eps_ref,
             mu_out, lv_out, z_out, ra_out, rn_out):
        apv = ap_ref[...]
        nbpv = nbp_ref[...]
        wmun_v = wmun[...]
        wlvn_v = wlvn[...]
        mu = _dot(apv, wmua[...]) + bmu_ref[...]
        lv = _dot(apv, wlva[...]) + blv_ref[...]
        for m in range(M):
            mu += _dot(nbpv[:, m, :], wmun_v[m])
            lv += _dot(nbpv[:, m, :], wlvn_v[m])
        std = jnp.exp(0.5 * lv)
        z = mu + eps_ref[...] * std
        mu_out[...] = mu
        lv_out[...] = lv
        z_out[...] = z
        ra_out[...] = (_dot(z, wdeca[...]) + bdeca_ref[...]) / apv
        wdecn_v = wdecn[...]
        bdecn_v = bdecn_ref[...]
        parts = [(_dot(z, wdecn_v[m]) + bdecn_v[m]) / nbpv[:, m, :]
                 for m in range(M)]
        rn_out[...] = jnp.stack(parts, axis=1)

    args = (ap, nbp, WmuaT, Wmun, bmu, WlvaT, Wlvn, blv,
            WdecaT, Wdecn, bdeca, bdecn, eps)
    return pl.pallas_call(
        body,
        grid=(1,),
        in_specs=[_full_spec(x.shape) for x in args],
        out_specs=[_full_spec(s) for s in
                   [(C, L), (C, L), (C, L), (C, A), (C, M, Bb)]],
        out_shape=[
            jax.ShapeDtypeStruct((C, L), F32),
            jax.ShapeDtypeStruct((C, L), F32),
            jax.ShapeDtypeStruct((C, L), F32),
            jax.ShapeDtypeStruct((C, A), F32),
            jax.ShapeDtypeStruct((C, M, Bb), F32),
        ],
        compiler_params=_ARB,
    )(*args)


def _td(a, nb, ra, rn, APC, TCc):
    N = a.shape[0]
    _, M, Bb = nb.shape
    C, A = ra.shape
    ra3 = ra.reshape(C, 1, A)
    T = TCc * APC
    G = C // TCc

    def body(a_ref, nb_ref, ra_ref, rn_ref, za_out, znb_out):
        rab = jnp.broadcast_to(ra_ref[...], (TCc, APC, A)).reshape(T, A)
        za_out[...] = _padw(_softplus(a_ref[...][:, :A] * rab), T)
        rnb = jnp.broadcast_to(rn_ref[...][:, None, :, :],
                               (TCc, APC, M, Bb)).reshape(T, M, Bb)
        znb_out[...] = _softplus(nb_ref[...] * rnb)

    return pl.pallas_call(
        body,
        grid=(G,),
        in_specs=[
            pl.BlockSpec((T, PW), lambda i: (i, 0)),
            pl.BlockSpec((T, M, Bb), lambda i: (i, 0, 0)),
            pl.BlockSpec((TCc, 1, A), lambda i: (i, 0, 0)),
            pl.BlockSpec((TCc, M, Bb), lambda i: (i, 0, 0)),
        ],
        out_specs=[
            pl.BlockSpec((T, PW), lambda i: (i, 0)),
            pl.BlockSpec((T, M, Bb), lambda i: (i, 0, 0)),
        ],
        out_shape=[
            jax.ShapeDtypeStruct((N, PW), F32),
            jax.ShapeDtypeStruct((N, M, Bb), F32),
        ],
        compiler_params=_ARB,
    )(a, nb, ra3, rn)


def _proj_sigmoid(a, WT, b, A, T):
    N = a.shape[0]
    G = N // T

    def body(a_ref, w_ref, b_ref, out_ref):
        out_ref[...] = _sigmoid(_dot(a_ref[...][:, :A], w_ref[...])
                                + b_ref[...])

    return pl.pallas_call(
        body,
        grid=(G,),
        in_specs=[
            pl.BlockSpec((T, PW), lambda i: (i, 0)),
            _full_spec(WT.shape),
            _full_spec(b.shape),
        ],
        out_specs=pl.BlockSpec((T, PW), lambda i: (i, 0)),
        out_shape=jax.ShapeDtypeStruct((N, PW), F32),
        compiler_params=_ARB,
    )(a, WT, b)


def _assemble_final(zfin, znf, znb, WbT, bb, Dbase, T):
    N = zfin.shape[0]
    _, M, Bb = znb.shape
    Dx = WbT.shape[1]
    Dout = 2 * Dbase + Dx
    G = N // T

    def body(zf_ref, znf_ref, znb_ref, w_ref, b_ref, out_ref):
        zf = zf_ref[...][:, :Dbase]
        p0 = jnp.broadcast_to(zf[:, None, :], (T, M, Dbase))
        p1 = znf_ref[...].reshape(T, M, PW)[:, :, :Dbase]
        p2 = _sigmoid(_dot(znb_ref[...].reshape(T * M, Bb), w_ref[...])
                      + b_ref[...]).reshape(T, M, Dx)
        out_ref[...] = jnp.concatenate([p0, p1, p2], axis=2)

    return pl.pallas_call(
        body,
        grid=(G,),
        in_specs=[
            pl.BlockSpec((T, PW), lambda i: (i, 0)),
            pl.BlockSpec((T * M, PW), lambda i: (i, 0)),
            pl.BlockSpec((T, M, Bb), lambda i: (i, 0, 0)),
            _full_spec(WbT.shape),
            _full_spec(bb.shape),
        ],
        out_specs=pl.BlockSpec((T, M, Dout), lambda i: (i, 0, 0)),
        out_shape=jax.ShapeDtypeStruct((N, M, Dout), F32),
        compiler_params=_ARB,
    )(zfin, znf, znb, WbT, bb)


def _split_conv_weights(W1, W2, A, Bb):
    Wcat = jnp.concatenate([W1.T, W2.T], axis=1)
    return [Wcat[:A], Wcat[A:2 * A], Wcat[2 * A:]]


def _conv_layer(a, nb, idxf, W, T):
    an = _sc_gather(a, idxf)
    stats = _conv_passA(a, an, nb, W, T)
    nb2, asum, s1a, s2a = _conv_passB(a, an, nb, W, stats, T)
    a2 = _conv_passC(a, asum, s1a, s2a, T)
    return a2, nb2


def kernel(atom_fea, nbr_fea, We_atom, be_atom, We_bond, be_bond, We_atom2,
           be_atom2, We_bond2, be_bond2, W_mu, b_mu, W_lv, b_lv, W_dec, b_dec,
           Wc1, Wc2, Wd1, Wd2, nbr_fea_idx, crystal_atom_idx):
    N, DA = atom_fea.shape
    _, M, DB = nbr_fea.shape
    A = We_atom.shape[0]
    Bb = We_bond.shape[0]
    C, APC = crystal_atom_idx.shape
    L = W_mu.shape[0]
    NCl = Wc1.shape[0]

    idxf = nbr_fea_idx.reshape(N * M).astype(jnp.int32)
    T1 = 400
    TE = 1000
    TA = 400

    af_pad = jnp.concatenate([atom_fea, jnp.zeros((N, PW - DA), F32)], axis=1)
    an0 = _sc_gather(af_pad, idxf)
    total_input_fea = _assemble(atom_fea, an0, nbr_fea, TA)

    a, nb = _embed(atom_fea, nbr_fea, We_atom.T, be_atom.reshape(1, A),
                   We_bond.T, be_bond.reshape(1, Bb), TE)

    for i in range(NCl):
        W = _split_conv_weights(Wc1[i], Wc2[i], A, Bb)
        a, nb = _conv_layer(a, nb, idxf, W, T1)

    ap = _pool_a(a, APC, A)
    nbp = _pool_nb(nb, APC, 25)
    eps = jax.random.normal(jax.random.key(42), (C, L), F32)
    Wmun = W_mu[:, A:].T.reshape(M, Bb, L)
    Wlvn = W_lv[:, A:].T.reshape(M, Bb, L)
    Wdecn = W_dec[A:, :].reshape(M, Bb, L).transpose(0, 2, 1)
    bdecn = b_dec[A:].reshape(M, 1, Bb)
    mu, logvar, z, ra, rn = _vae(
        ap, nbp, W_mu[:, :A].T, Wmun, b_mu.reshape(1, L),
        W_lv[:, :A].T, Wlvn, b_lv.reshape(1, L),
        W_dec[:A, :].T, Wdecn, b_dec[:A].reshape(1, A), bdecn, eps)

    za, znb = _td(a, nb, ra, rn, APC, 25)

    for i in range(NCl):
        W = _split_conv_weights(Wd1[i], Wd2[i], A, Bb)
        za, znb = _conv_layer(za, znb, idxf, W, T1)

    Wa2T_pad = jnp.concatenate(
        [We_atom2, jnp.zeros((PW - DA, A), F32)], axis=0).T
    ba2_pad = jnp.concatenate([be_atom2, jnp.zeros((PW - DA,), F32)]) \
        .reshape(1, PW)
    zfin = _proj_sigmoid(za, Wa2T_pad, ba2_pad, A, TE)
    znf = _sc_gather(zfin, idxf)
    z_decoded = _assemble_final(zfin, znf, znb, We_bond2.T,
                                be_bond2.reshape(1, DB), DA, TA)

    return (z_decoded, mu, logvar, z, total_input_fea)


# speedup vs baseline: 1.0035x; 1.0035x over previous
import functools

import jax
import jax.numpy as jnp
from jax import lax
from jax.experimental import pallas as pl
from jax.experimental.pallas import tpu as pltpu
from jax.experimental.pallas import tpu_sc as plsc

F32 = jnp.float32
PW = 128


def _softplus(x):
    return jnp.maximum(x, 0.0) + jnp.log(1.0 + jnp.exp(-jnp.abs(x)))


def _sigmoid(x):
    return 1.0 / (1.0 + jnp.exp(-x))


def _dot(x, w):
    return lax.dot_general(x, w, (((1,), (0,)), ((), ())),
                           preferred_element_type=F32)


def _padw(x, T):
    K = x.shape[1]
    return jnp.concatenate([x, jnp.zeros((T, PW - K), F32)], axis=1)


def _sc_gather(table, idx):
    V, D = table.shape
    B = idx.shape[0]
    info = plsc.get_sparse_core_info()
    ncores = info.num_cores
    nw = ncores * info.num_subcores
    bw = B // nw
    ch = 200 if bw % 200 == 0 else bw
    nchunk = bw // ch
    npair = (nchunk - 1) // 2
    tail = nchunk - 1 - 2 * npair
    mesh = plsc.VectorSubcoreMesh(core_axis_name="c", subcore_axis_name="s")

    @functools.partial(
        pl.kernel, mesh=mesh,
        out_type=jax.ShapeDtypeStruct((B, D), F32),
        scratch_types=[pltpu.VMEM((ch,), jnp.int32),
                       pltpu.VMEM((ch,), jnp.int32),
                       pltpu.VMEM((ch, D), F32),
                       pltpu.VMEM((ch, D), F32),
                       pltpu.SemaphoreType.DMA,
                       pltpu.SemaphoreType.DMA],
    )
    def k(table_hbm, idx_hbm, out_hbm, idx0, idx1, rows0, rows1, sem0, sem1):
        wid = lax.axis_index("s") * ncores + lax.axis_index("c")
        base = wid * bw

        def start(c, idx_v, rows_v, sem):
            pltpu.sync_copy(idx_hbm.at[pl.ds(base + c * ch, ch)], idx_v)
            pltpu.async_copy(table_hbm.at[idx_v], rows_v, sem)

        def drain(c, rows_v, sem):
            pltpu.make_async_copy(table_hbm.at[pl.ds(0, ch)], rows_v,
                                  sem).wait()
            pltpu.sync_copy(rows_v, out_hbm.at[pl.ds(base + c * ch, ch)])

        start(0, idx0, rows0, sem0)

        def body(p, carry):
            c1 = 2 * p + 1
            start(c1, idx1, rows1, sem1)
            drain(c1 - 1, rows0, sem0)
            start(c1 + 1, idx0, rows0, sem0)
            drain(c1, rows1, sem1)
            return carry

        lax.fori_loop(0, npair, body, 0)
        if tail:
            start(nchunk - 1, idx1, rows1, sem1)
            drain(nchunk - 2, rows0, sem0)
            drain(nchunk - 1, rows1, sem1)
        else:
            drain(nchunk - 1, rows0, sem0)

    return k(table, idx)


_ARB = pltpu.CompilerParams(dimension_semantics=("arbitrary",))


def _full_spec(shape):
    rank = len(shape)
    return pl.BlockSpec(shape, lambda i, _r=rank: (0,) * _r)


def _embed(atom_fea, nbr_fea, WeaT, bea, WebT, beb, T):
    N, DA = atom_fea.shape
    _, M, DB = nbr_fea.shape
    A = WeaT.shape[1]
    Bb = WebT.shape[1]
    G = N // T

    def body(af_ref, nbr_ref, wa_ref, ba_ref, wb_ref, bb_ref, a_out, nb_out):
        a_out[...] = _padw(_dot(af_ref[...], wa_ref[...]) + ba_ref[...], T)
        nbf = nbr_ref[...].reshape(T * M, DB)
        nb = _dot(nbf, wb_ref[...]) + bb_ref[...]
        nb_out[...] = nb.reshape(T, M, Bb)

    return pl.pallas_call(
        body,
        grid=(G,),
        in_specs=[
            pl.BlockSpec((T, DA), lambda i: (i, 0)),
            pl.BlockSpec((T, M, DB), lambda i: (i, 0, 0)),
            _full_spec(WeaT.shape),
            _full_spec(bea.shape),
            _full_spec(WebT.shape),
            _full_spec(beb.shape),
        ],
        out_specs=[
            pl.BlockSpec((T, PW), lambda i: (i, 0)),
            pl.BlockSpec((T, M, Bb), lambda i: (i, 0, 0)),
        ],
        out_shape=[
            jax.ShapeDtypeStruct((N, PW), F32),
            jax.ShapeDtypeStruct((N, M, Bb), F32),
        ],
        compiler_params=_ARB,
    )(atom_fea, nbr_fea, WeaT, bea, WebT, beb)


def _assemble(base, gath, extra, T):
    N, Dbase = base.shape
    _, M, Dx = extra.shape
    Dout = 2 * Dbase + Dx
    G = N // T

    def body(b_ref, g_ref, x_ref, out_ref):
        b = b_ref[...]
        p0 = jnp.broadcast_to(b[:, None, :], (T, M, Dbase))
        p1 = g_ref[...].reshape(T, M, PW)[:, :, :Dbase]
        out_ref[...] = jnp.concatenate([p0, p1, x_ref[...]], axis=2)

    return pl.pallas_call(
        body,
        grid=(G,),
        in_specs=[
            pl.BlockSpec((T, Dbase), lambda i: (i, 0)),
            pl.BlockSpec((T * M, PW), lambda i: (i, 0)),
            pl.BlockSpec((T, M, Dx), lambda i: (i, 0, 0)),
        ],
        out_specs=pl.BlockSpec((T, M, Dout), lambda i: (i, 0, 0)),
        out_shape=jax.ShapeDtypeStruct((N, M, Dout), F32),
        compiler_params=_ARB,
    )(base, gath, extra)


def _conv_passA(a, an, nb, W, T):
    N = a.shape[0]
    _, M, Bb = nb.shape
    A = W[0].shape[0]
    P = W[0].shape[1]
    G = N // T

    def body(a_ref, an_ref, nb_ref, wa, wn, wb, s1, s2):
        i = pl.program_id(0)
        base = _dot(a_ref[...][:, :A], wa[...])
        baseb = jnp.broadcast_to(base[:, None, :], (T, M, P)).reshape(T * M, P)
        an_t = an_ref[...][:, :A]
        nbf = nb_ref[...].reshape(T * M, Bb)
        raw = baseb + _dot(an_t, wn[...]) + _dot(nbf, wb[...])

        @pl.when(i == 0)
        def _():
            s1[...] = jnp.zeros_like(s1)
            s2[...] = jnp.zeros_like(s2)

        s1[...] += jnp.sum(raw, axis=0, keepdims=True)
        s2[...] += jnp.sum(raw * raw, axis=0, keepdims=True)

    return pl.pallas_call(
        body,
        grid=(G,),
        in_specs=[
            pl.BlockSpec((T, PW), lambda i: (i, 0)),
            pl.BlockSpec((T * M, PW), lambda i: (i, 0)),
            pl.BlockSpec((T, M, Bb), lambda i: (i, 0, 0)),
        ] + [_full_spec(w.shape) for w in W],
        out_specs=[_full_spec((1, P)), _full_spec((1, P))],
        out_shape=[jax.ShapeDtypeStruct((1, P), F32),
                   jax.ShapeDtypeStruct((1, P), F32)],
        compiler_params=_ARB,
    )(a, an, nb, *W)


def _conv_passB(a, an, nb, W, stats, T):
    N = a.shape[0]
    _, M, Bb = nb.shape
    A = W[0].shape[0]
    P = W[0].shape[1]
    G = N // T
    R = float(N * M)

    def body(a_ref, an_ref, nb_ref, wa, wn, wb, s1, s2,
             nb_out, asum_out, s1a, s2a):
        i = pl.program_id(0)
        base = _dot(a_ref[...][:, :A], wa[...])
        baseb = jnp.broadcast_to(base[:, None, :], (T, M, P)).reshape(T * M, P)
        an_t = an_ref[...][:, :A]
        nb_t = nb_ref[...].reshape(T * M, Bb)
        raw = baseb + _dot(an_t, wn[...]) + _dot(nb_t, wb[...])

        m = s1[...] / R
        v = s2[...] / R - m * m
        rawn = (raw - m) * lax.rsqrt(v + 1e-5)

        g = _sigmoid(rawn[:, :A]) * _softplus(rawn[:, A:2 * A])
        asum_t = jnp.sum(g.reshape(T, M, A), axis=1)
        asum_out[...] = asum_t
        h = _sigmoid(rawn[:, 2 * A:2 * A + Bb]) * _softplus(rawn[:, 2 * A + Bb:])
        nb_out[...] = _softplus(nb_t + h).reshape(T, M, Bb)

        @pl.when(i == 0)
        def _():
            s1a[...] = jnp.zeros_like(s1a)
            s2a[...] = jnp.zeros_like(s2a)

        s1a[...] += jnp.sum(asum_t, axis=0, keepdims=True)
        s2a[...] += jnp.sum(asum_t * asum_t, axis=0, keepdims=True)

    return pl.pallas_call(
        body,
        grid=(G,),
        in_specs=[
            pl.BlockSpec((T, PW), lambda i: (i, 0)),
            pl.BlockSpec((T * M, PW), lambda i: (i, 0)),
            pl.BlockSpec((T, M, Bb), lambda i: (i, 0, 0)),
        ] + [_full_spec(w.shape) for w in W]
          + [_full_spec(s.shape) for s in stats],
        out_specs=[
            pl.BlockSpec((T, M, Bb), lambda i: (i, 0, 0)),
            pl.BlockSpec((T, A), lambda i: (i, 0)),
            _full_spec((1, A)),
            _full_spec((1, A)),
        ],
        out_shape=[
            jax.ShapeDtypeStruct((N, M, Bb), F32),
            jax.ShapeDtypeStruct((N, A), F32),
            jax.ShapeDtypeStruct((1, A), F32),
            jax.ShapeDtypeStruct((1, A), F32),
        ],
        compiler_params=_ARB,
    )(a, an, nb, *W, *stats)


def _conv_passC(a, asum, s1a, s2a, T):
    N = a.shape[0]
    A = asum.shape[1]
    G = N // T
    R = float(N)

    def body(a_ref, as_ref, s1_ref, s2_ref, out_ref):
        m = s1_ref[...] / R
        v = s2_ref[...] / R - m * m
        out = _softplus(a_ref[...][:, :A]
                        + (as_ref[...] - m) * lax.rsqrt(v + 1e-5))
        out_ref[...] = _padw(out, T)

    return pl.pallas_call(
        body,
        grid=(G,),
        in_specs=[
            pl.BlockSpec((T, PW), lambda i: (i, 0)),
            pl.BlockSpec((T, A), lambda i: (i, 0)),
            _full_spec((1, A)),
            _full_spec((1, A)),
        ],
        out_specs=pl.BlockSpec((T, PW), lambda i: (i, 0)),
        out_shape=jax.ShapeDtypeStruct((N, PW), F32),
        compiler_params=_ARB,
    )(a, asum, s1a, s2a)


def _pool_a(a, APC, A):
    N = a.shape[0]
    C = N // APC

    def body(a_ref, ap_out):
        at = a_ref[...][:, :A].reshape(C, APC, A)
        ap_out[...] = _softplus(jnp.mean(at, axis=1))

    return pl.pallas_call(
        body,
        grid=(1,),
        in_specs=[_full_spec((N, PW))],
        out_specs=_full_spec((C, A)),
        out_shape=jax.ShapeDtypeStruct((C, A), F32),
        compiler_params=_ARB,
    )(a)


def _pool_nb(nb, APC, TCc):
    N, M, Bb = nb.shape
    C = N // APC
    nb4 = nb.reshape(C, APC, M, Bb)
    G = C // TCc

    def body(nb_ref, nbp_out):
        nbp_out[...] = _softplus(jnp.mean(nb_ref[...], axis=1))

    return pl.pallas_call(
        body,
        grid=(G,),
        in_specs=[pl.BlockSpec((TCc, APC, M, Bb), lambda i: (i, 0, 0, 0))],
        out_specs=pl.BlockSpec((TCc, M, Bb), lambda i: (i, 0, 0)),
        out_shape=jax.ShapeDtypeStruct((C, M, Bb), F32),
        compiler_params=_ARB,
    )(nb4)


def _vae(ap, nbp, WmuaT, Wmun, bmu, WlvaT, Wlvn, blv,
         WdecaT, Wdecn, bdeca, bdecn, eps):
    C, A = ap.shape
    _, M, Bb = nbp.shape
    L = WmuaT.shape[1]

    def body(ap_ref, nbp_ref, wmua, wmun, bmu_ref, wlva, wlvn, blv_ref,
             wdeca, wdecn, bdeca_ref, bdecn_ref, eps_ref,
             mu_out, lv_out, z_out, ra_out, rn_out):
        apv = ap_ref[...]
        nbpv = nbp_ref[...]
        wmun_v = wmun[...]
        wlvn_v = wlvn[...]
        mu = _dot(apv, wmua[...]) + bmu_ref[...]
        lv = _dot(apv, wlva[...]) + blv_ref[...]
        for m in range(M):
            mu += _dot(nbpv[:, m, :], wmun_v[m])
            lv += _dot(nbpv[:, m, :], wlvn_v[m])
        std = jnp.exp(0.5 * lv)
        z = mu + eps_ref[...] * std
        mu_out[...] = mu
        lv_out[...] = lv
        z_out[...] = z
        ra_out[...] = (_dot(z, wdeca[...]) + bdeca_ref[...]) / apv
        wdecn_v = wdecn[...]
        bdecn_v = bdecn_ref[...]
        parts = [(_dot(z, wdecn_v[m]) + bdecn_v[m]) / nbpv[:, m, :]
                 for m in range(M)]
        rn_out[...] = jnp.stack(parts, axis=1)

    args = (ap, nbp, WmuaT, Wmun, bmu, WlvaT, Wlvn, blv,
            WdecaT, Wdecn, bdeca, bdecn, eps)
    return pl.pallas_call(
        body,
        grid=(1,),
        in_specs=[_full_spec(x.shape) for x in args],
        out_specs=[_full_spec(s) for s in
                   [(C, L), (C, L), (C, L), (C, A), (C, M, Bb)]],
        out_shape=[
            jax.ShapeDtypeStruct((C, L), F32),
            jax.ShapeDtypeStruct((C, L), F32),
            jax.ShapeDtypeStruct((C, L), F32),
            jax.ShapeDtypeStruct((C, A), F32),
            jax.ShapeDtypeStruct((C, M, Bb), F32),
        ],
        compiler_params=_ARB,
    )(*args)


def _td(a, nb, ra, rn, APC, TCc):
    N = a.shape[0]
    _, M, Bb = nb.shape
    C, A = ra.shape
    ra3 = ra.reshape(C, 1, A)
    T = TCc * APC
    G = C // TCc

    def body(a_ref, nb_ref, ra_ref, rn_ref, za_out, znb_out):
        rab = jnp.broadcast_to(ra_ref[...], (TCc, APC, A)).reshape(T, A)
        za_out[...] = _padw(_softplus(a_ref[...][:, :A] * rab), T)
        rnb = jnp.broadcast_to(rn_ref[...][:, None, :, :],
                               (TCc, APC, M, Bb)).reshape(T, M, Bb)
        znb_out[...] = _softplus(nb_ref[...] * rnb)

    return pl.pallas_call(
        body,
        grid=(G,),
        in_specs=[
            pl.BlockSpec((T, PW), lambda i: (i, 0)),
            pl.BlockSpec((T, M, Bb), lambda i: (i, 0, 0)),
            pl.BlockSpec((TCc, 1, A), lambda i: (i, 0, 0)),
            pl.BlockSpec((TCc, M, Bb), lambda i: (i, 0, 0)),
        ],
        out_specs=[
            pl.BlockSpec((T, PW), lambda i: (i, 0)),
            pl.BlockSpec((T, M, Bb), lambda i: (i, 0, 0)),
        ],
        out_shape=[
            jax.ShapeDtypeStruct((N, PW), F32),
            jax.ShapeDtypeStruct((N, M, Bb), F32),
        ],
        compiler_params=_ARB,
    )(a, nb, ra3, rn)


def _proj_sigmoid(a, WT, b, A, T):
    N = a.shape[0]
    G = N // T

    def body(a_ref, w_ref, b_ref, out_ref):
        out_ref[...] = _sigmoid(_dot(a_ref[...][:, :A], w_ref[...])
                                + b_ref[...])

    return pl.pallas_call(
        body,
        grid=(G,),
        in_specs=[
            pl.BlockSpec((T, PW), lambda i: (i, 0)),
            _full_spec(WT.shape),
            _full_spec(b.shape),
        ],
        out_specs=pl.BlockSpec((T, PW), lambda i: (i, 0)),
        out_shape=jax.ShapeDtypeStruct((N, PW), F32),
        compiler_params=_ARB,
    )(a, WT, b)


def _assemble_final(zfin, znf, znb, WbT, bb, Dbase, T):
    N = zfin.shape[0]
    _, M, Bb = znb.shape
    Dx = WbT.shape[1]
    Dout = 2 * Dbase + Dx
    G = N // T

    def body(zf_ref, znf_ref, znb_ref, w_ref, b_ref, out_ref):
        zf = zf_ref[...][:, :Dbase]
        p0 = jnp.broadcast_to(zf[:, None, :], (T, M, Dbase))
        p1 = znf_ref[...].reshape(T, M, PW)[:, :, :Dbase]
        p2 = _sigmoid(_dot(znb_ref[...].reshape(T * M, Bb), w_ref[...])
                      + b_ref[...]).reshape(T, M, Dx)
        out_ref[...] = jnp.concatenate([p0, p1, p2], axis=2)

    return pl.pallas_call(
        body,
        grid=(G,),
        in_specs=[
            pl.BlockSpec((T, PW), lambda i: (i, 0)),
            pl.BlockSpec((T * M, PW), lambda i: (i, 0)),
            pl.BlockSpec((T, M, Bb), lambda i: (i, 0, 0)),
            _full_spec(WbT.shape),
            _full_spec(bb.shape),
        ],
        out_specs=pl.BlockSpec((T, M, Dout), lambda i: (i, 0, 0)),
        out_shape=jax.ShapeDtypeStruct((N, M, Dout), F32),
        compiler_params=_ARB,
    )(zfin, znf, znb, WbT, bb)


def _split_conv_weights(W1, W2, A, Bb):
    Wcat = jnp.concatenate([W1.T, W2.T], axis=1)
    return [Wcat[:A], Wcat[A:2 * A], Wcat[2 * A:]]


def _conv_layer(a, nb, idxf, W, T):
    an = _sc_gather(a, idxf)
    stats = _conv_passA(a, an, nb, W, T)
    nb2, asum, s1a, s2a = _conv_passB(a, an, nb, W, stats, T)
    a2 = _conv_passC(a, asum, s1a, s2a, T)
    return a2, nb2


def kernel(atom_fea, nbr_fea, We_atom, be_atom, We_bond, be_bond, We_atom2,
           be_atom2, We_bond2, be_bond2, W_mu, b_mu, W_lv, b_lv, W_dec, b_dec,
           Wc1, Wc2, Wd1, Wd2, nbr_fea_idx, crystal_atom_idx):
    N, DA = atom_fea.shape
    _, M, DB = nbr_fea.shape
    A = We_atom.shape[0]
    Bb = We_bond.shape[0]
    C, APC = crystal_atom_idx.shape
    L = W_mu.shape[0]
    NCl = Wc1.shape[0]

    idxf = nbr_fea_idx.reshape(N * M).astype(jnp.int32)
    T1 = 1000
    TE = 1000
    TA = 400

    af_pad = jnp.concatenate([atom_fea, jnp.zeros((N, PW - DA), F32)], axis=1)
    an0 = _sc_gather(af_pad, idxf)
    total_input_fea = _assemble(atom_fea, an0, nbr_fea, TA)

    a, nb = _embed(atom_fea, nbr_fea, We_atom.T, be_atom.reshape(1, A),
                   We_bond.T, be_bond.reshape(1, Bb), TE)

    for i in range(NCl):
        W = _split_conv_weights(Wc1[i], Wc2[i], A, Bb)
        a, nb = _conv_layer(a, nb, idxf, W, T1)

    ap = _pool_a(a, APC, A)
    nbp = _pool_nb(nb, APC, 25)
    eps = jax.random.normal(jax.random.key(42), (C, L), F32)
    Wmun = W_mu[:, A:].T.reshape(M, Bb, L)
    Wlvn = W_lv[:, A:].T.reshape(M, Bb, L)
    Wdecn = W_dec[A:, :].reshape(M, Bb, L).transpose(0, 2, 1)
    bdecn = b_dec[A:].reshape(M, 1, Bb)
    mu, logvar, z, ra, rn = _vae(
        ap, nbp, W_mu[:, :A].T, Wmun, b_mu.reshape(1, L),
        W_lv[:, :A].T, Wlvn, b_lv.reshape(1, L),
        W_dec[:A, :].T, Wdecn, b_dec[:A].reshape(1, A), bdecn, eps)

    za, znb = _td(a, nb, ra, rn, APC, 25)

    for i in range(NCl):
        W = _split_conv_weights(Wd1[i], Wd2[i], A, Bb)
        za, znb = _conv_layer(za, znb, idxf, W, T1)

    Wa2T_pad = jnp.concatenate(
        [We_atom2, jnp.zeros((PW - DA, A), F32)], axis=0).T
    ba2_pad = jnp.concatenate([be_atom2, jnp.zeros((PW - DA,), F32)]) \
        .reshape(1, PW)
    zfin = _proj_sigmoid(za, Wa2T_pad, ba2_pad, A, TE)
    znf = _sc_gather(zfin, idxf)
    z_decoded = _assemble_final(zfin, znf, znb, We_bond2.T,
                                be_bond2.reshape(1, DB), DA, TA)

    return (z_decoded, mu, logvar, z, total_input_fea)
